# Initial kernel scaffold; baseline (speedup 1.0000x reference)
import jax
import jax.numpy as jnp
from jax.experimental import pallas as pl


def kernel(input_ids, embedding_weight):
    raise NotImplementedError("write your pallas kernel here")


# speedup vs baseline: 1.4378x; 1.4378x over previous
import functools

import jax
import jax.numpy as jnp
from jax import lax
from jax.experimental import pallas as pl
from jax.experimental.pallas import tpu as pltpu
from jax.experimental.pallas import tpu_sc as plsc

_NC = 2
_NS = 16
_NW = _NC * _NS
_K = 32


def _build(b_flat, d):
    rows_per_w = b_flat // _NW
    nchunks = rows_per_w // _K
    mesh = plsc.VectorSubcoreMesh(core_axis_name="c", subcore_axis_name="s")

    @functools.partial(
        pl.kernel,
        mesh=mesh,
        out_type=jax.ShapeDtypeStruct((b_flat, d), jnp.float32),
        scratch_types=[
            pltpu.VMEM((nchunks, _K), jnp.int32),
            pltpu.VMEM((_K, d), jnp.float32),
            pltpu.SemaphoreType.DMA,
        ],
    )
    def k(ids_hbm, table_hbm, out_hbm, idx_v, buf, sem):
        wid = lax.axis_index("s") * _NC + lax.axis_index("c")
        base = wid * rows_per_w
        pltpu.sync_copy(ids_hbm.at[wid], idx_v)

        def body(ch, carry):
            pltpu.async_copy(table_hbm.at[idx_v.at[ch]], buf, sem).wait()
            pltpu.sync_copy(buf, out_hbm.at[pl.ds(base + ch * _K, _K)])
            return carry

        lax.fori_loop(0, nchunks, body, 0)

    return k


def kernel(input_ids, embedding_weight):
    bt, s = input_ids.shape
    b_flat = bt * s
    d = embedding_weight.shape[1]
    ids3 = input_ids.reshape(_NW, b_flat // _NW // _K, _K).astype(jnp.int32)
    out = _build(b_flat, d)(ids3, embedding_weight)
    return out.reshape(bt, s, d)


# speedup vs baseline: 1.6636x; 1.1570x over previous
import functools

import jax
import jax.numpy as jnp
from jax import lax
from jax.experimental import pallas as pl
from jax.experimental.pallas import tpu as pltpu
from jax.experimental.pallas import tpu_sc as plsc

_NC = 2
_NS = 16
_NW = _NC * _NS
_K = 32


def _build(b_flat, d):
    rows_per_w = b_flat // _NW
    nchunks = rows_per_w // _K
    mesh = plsc.VectorSubcoreMesh(core_axis_name="c", subcore_axis_name="s")

    npairs = nchunks // 2

    @functools.partial(
        pl.kernel,
        mesh=mesh,
        out_type=jax.ShapeDtypeStruct((b_flat, d), jnp.float32),
        scratch_types=[
            pltpu.VMEM((nchunks, _K), jnp.int32),
            pltpu.VMEM((_K, d), jnp.float32),
            pltpu.VMEM((_K, d), jnp.float32),
            pltpu.SemaphoreType.DMA,
            pltpu.SemaphoreType.DMA,
        ],
    )
    def k(ids_hbm, table_hbm, out_hbm, idx_v, buf0, buf1, sem0, sem1):
        wid = lax.axis_index("s") * _NC + lax.axis_index("c")
        base = wid * rows_per_w
        pltpu.sync_copy(ids_hbm.at[wid], idx_v)

        bufs = (buf0, buf1)
        sems = (sem0, sem1)

        pltpu.async_copy(table_hbm.at[idx_v.at[0]], buf0, sem0)
        pltpu.async_copy(table_hbm.at[idx_v.at[1]], buf1, sem1)

        def body(p, carry):
            ch = p * 2
            for b in range(2):
                c = ch + b
                pltpu.make_async_copy(
                    table_hbm.at[idx_v.at[c]], bufs[b], sems[b]
                ).wait()
                pltpu.sync_copy(bufs[b], out_hbm.at[pl.ds(base + c * _K, _K)])
                pltpu.async_copy(table_hbm.at[idx_v.at[c + 2]], bufs[b], sems[b])
            return carry

        lax.fori_loop(0, npairs - 1, body, 0)

        for b in range(2):
            c = (npairs - 1) * 2 + b
            pltpu.make_async_copy(
                table_hbm.at[idx_v.at[c]], bufs[b], sems[b]
            ).wait()
            pltpu.sync_copy(bufs[b], out_hbm.at[pl.ds(base + c * _K, _K)])

    return k


def kernel(input_ids, embedding_weight):
    bt, s = input_ids.shape
    b_flat = bt * s
    d = embedding_weight.shape[1]
    ids3 = input_ids.reshape(_NW, b_flat // _NW // _K, _K).astype(jnp.int32)
    out = _build(b_flat, d)(ids3, embedding_weight)
    return out.reshape(bt, s, d)
